# baseline (speedup 1.0000x reference)
import functools

import jax
import jax.numpy as jnp
from jax import lax
from jax.experimental import pallas as pl
from jax.experimental.pallas import tpu as pltpu
from jax.experimental.pallas import tpu_sc as plsc

N_ENT = 50000
NUM_POS_REL = 8
BODY_LEN = 3
B = 64
E = 200000
THR = 1e-20

L = 16
NW = 32
C1 = 10000
C2 = 4000
ATTP = 64

_mesh = plsc.VectorSubcoreMesh(core_axis_name="c", subcore_axis_name="s")


def _wid():
    return lax.axis_index("s") * 2 + lax.axis_index("c")


@functools.partial(
    pl.kernel,
    out_type=(
        jax.ShapeDtypeStruct((NUM_POS_REL * E,), jnp.int32),
        jax.ShapeDtypeStruct((NUM_POS_REL * E,), jnp.float32),
    ),
    mesh=_mesh,
    compiler_params=pltpu.CompilerParams(needs_layout_passes=False),
    scratch_types=[
        pltpu.VMEM((B,), jnp.int32),
        pltpu.VMEM((B,), jnp.int32),
        pltpu.VMEM((B,), jnp.int32),
        pltpu.VMEM((B + 2 * L,), jnp.int32),
        pltpu.VMEM((C1,), jnp.int32),
        pltpu.VMEM((C1,), jnp.int32),
        pltpu.VMEM((C1,), jnp.float32),
        pltpu.VMEM((C1,), jnp.int32),
        pltpu.VMEM((C1,), jnp.float32),
        pltpu.VMEM((L,), jnp.int32),
    ],
)
def _mask_pack(ei_hbm, ev_hbm, qh_hbm, qr_hbm, tt_hbm, pk_out, mv_out,
               qh_v, qr_v, tt_v, keyb, srcb, dstb, valb, pkob, mvob, hitb):
    w = _wid()
    rel = w // 4
    quarter = w % 4
    epw = E // 4
    ebase = quarter * epw
    pltpu.sync_copy(qh_hbm, qh_v)
    pltpu.sync_copy(qr_hbm, qr_v)
    pltpu.sync_copy(tt_hbm, tt_v)

    for g in range((B + 2 * L) // L):
        keyb[pl.ds(g * L, L)] = jnp.full((L,), -1, jnp.int32)

    def build(g, cnt):
        qh = qh_v[pl.ds(g * L, L)]
        qr = qr_v[pl.ds(g * L, L)]
        tt = tt_v[pl.ds(g * L, L)]
        valid = qr == rel
        key = jnp.bitwise_or(qh, lax.shift_left(tt, 16))
        vi = valid.astype(jnp.int32)
        pos = cnt + lax.cumsum(vi, axis=0) - 1
        plsc.store_scatter(keyb, [pos], key, mask=valid)
        return cnt + jnp.sum(vi)

    nq = lax.fori_loop(0, B // L, build, jnp.int32(0))

    def chunk(c, _):
        off = ebase + c * C1
        pltpu.sync_copy(ei_hbm.at[pl.ds(rel * 2 * E + off, C1)], srcb)
        pltpu.sync_copy(ei_hbm.at[pl.ds(rel * 2 * E + E + off, C1)], dstb)
        pltpu.sync_copy(ev_hbm.at[pl.ds(rel * E + off, C1)], valb)

        def grp(g, _):
            sl = pl.ds(g * L, L)
            s = srcb[sl]
            d = dstb[sl]
            v = valb[sl]
            pk = jnp.bitwise_or(s, lax.shift_left(d, 16))

            hitb[...] = jnp.zeros((L,), jnp.int32)

            def keyloop(j, _):
                kj = plsc.load_gather(keyb, [jnp.full((L,), j, jnp.int32)])
                hitb[...] = hitb[...] | (pk == kj).astype(jnp.int32)
                return 0

            lax.fori_loop(0, nq, keyloop, 0)
            pkob[sl] = pk
            mvob[sl] = jnp.where(hitb[...] != 0, jnp.float32(0.0), v)
            return 0

        lax.fori_loop(0, C1 // L, grp, 0)
        pltpu.sync_copy(pkob, pk_out.at[pl.ds(rel * E + off, C1)])
        pltpu.sync_copy(mvob, mv_out.at[pl.ds(rel * E + off, C1)])
        return 0

    lax.fori_loop(0, epw // C1, chunk, 0)


@functools.partial(
    pl.kernel,
    out_type=(
        jax.ShapeDtypeStruct((B * L,), jnp.float32),
        jax.ShapeDtypeStruct((B * L,), jnp.int32),
    ),
    mesh=_mesh,
    compiler_params=pltpu.CompilerParams(needs_layout_passes=False),
    scratch_types=[
        pltpu.VMEM((N_ENT,), jnp.float32),
        pltpu.VMEM((N_ENT,), jnp.float32),
        pltpu.VMEM((C2,), jnp.int32),
        pltpu.VMEM((C2,), jnp.float32),
        pltpu.VMEM((C2,), jnp.int32),
        pltpu.VMEM((C2,), jnp.float32),
        pltpu.VMEM((B,), jnp.int32),
        pltpu.VMEM((B,), jnp.int32),
        pltpu.VMEM((ATTP,), jnp.float32),
        pltpu.VMEM((L,), jnp.float32),
        pltpu.VMEM((L,), jnp.int32),
        pltpu.VMEM((L,), jnp.float32),
        pltpu.VMEM((L,), jnp.int32),
        pltpu.VMEM((L,), jnp.int32),
        pltpu.SemaphoreType.DMA,
        pltpu.SemaphoreType.DMA,
    ],
)
def _propagate(pk_hbm, mv_hbm, qh_hbm, tt_hbm, att_hbm, pt_out, rk_out,
               mem_a, mem_b, pkA, vA, pkB, vB, qh_v, tt_v, att_v,
               st_p, st_r, svb, gtb, eqb, semA, semB):
    w = _wid()
    iota = lax.iota(jnp.int32, L)
    lane0 = iota == 0
    nz = N_ENT // L
    nch = E // C2
    ng = C2 // L
    pltpu.sync_copy(qh_hbm, qh_v)
    pltpu.sync_copy(tt_hbm, tt_v)

    def start(r, c, pkb, vb, sem):
        pltpu.async_copy(pk_hbm.at[pl.ds(r * E + c * C2, C2)], pkb, sem)
        pltpu.async_copy(mv_hbm.at[pl.ds(r * E + c * C2, C2)], vb, sem)

    def wait(pkb, vb, sem):
        pltpu.make_async_copy(pk_hbm.at[pl.ds(0, C2)], pkb, sem).wait()
        pltpu.make_async_copy(mv_hbm.at[pl.ds(0, C2)], vb, sem).wait()

    for p in range(2):
        b = w + NW * p
        pltpu.sync_copy(att_hbm.at[pl.ds(b * ATTP, ATTP)], att_v)
        bvec = jnp.full((L,), 0, jnp.int32) + b
        hb = plsc.load_gather(qh_v, [bvec])
        tb = plsc.load_gather(tt_v, [bvec])

        @plsc.parallel_loop(0, nz, unroll=8)
        def zero_a(i):
            mem_a[pl.ds(i * L, L)] = jnp.zeros((L,), jnp.float32)
        plsc.store_scatter(mem_a, [hb], jnp.ones((L,), jnp.float32), mask=lane0)

        bufs = (mem_a, mem_b)
        for step in range(BODY_LEN):
            src = bufs[step % 2]
            dst = bufs[1 - step % 2]

            @plsc.parallel_loop(0, nz, unroll=8)
            def zero_d(i):
                dst[pl.ds(i * L, L)] = jnp.zeros((L,), jnp.float32)

            def rel_body(r, _):
                attF = plsc.load_gather(att_v, [jnp.full((L,), step * 17, jnp.int32) + r])
                attB = plsc.load_gather(att_v, [jnp.full((L,), step * 17 + 8, jnp.int32) + r])

                def process(pkb, vb):
                    @plsc.parallel_loop(0, ng, unroll=16)
                    def grp(g):
                        sl = pl.ds(g * L, L)
                        wv = pkb[sl]
                        vv = vb[sl]
                        sidx = jnp.bitwise_and(wv, jnp.int32(0xFFFF))
                        didx = lax.shift_right_logical(wv, 16)
                        xf = plsc.load_gather(src, [sidx])
                        plsc.addupdate_scatter(dst, [didx], xf * vv * attF)
                        xb = plsc.load_gather(src, [didx])
                        plsc.addupdate_scatter(dst, [sidx], xb * vv * attB)

                start(r, 0, pkA, vA, semA)

                def pair(i, _):
                    start(r, 2 * i + 1, pkB, vB, semB)
                    wait(pkA, vA, semA)
                    process(pkA, vA)

                    @pl.when(i < nch // 2 - 1)
                    def _():
                        start(r, 2 * i + 2, pkA, vA, semA)

                    wait(pkB, vB, semB)
                    process(pkB, vB)
                    return 0

                lax.fori_loop(0, nch // 2, pair, 0)
                return 0

            lax.fori_loop(0, NUM_POS_REL, rel_body, 0)

            attS = plsc.load_gather(att_v, [jnp.full((L,), step * 17 + 16, jnp.int32)])

            @plsc.parallel_loop(0, nz, unroll=8)
            def selfadd(i):
                sl = pl.ds(i * L, L)
                dst[sl] = dst[sl] + src[sl] * attS

        fin = bufs[BODY_LEN % 2]
        mt = plsc.load_gather(fin, [tb])

        svb[...] = jnp.zeros((L,), jnp.float32)
        gtb[...] = jnp.zeros((L,), jnp.int32)
        eqb[...] = jnp.zeros((L,), jnp.int32)

        def red(i, _):
            m = fin[pl.ds(i * L, L)]
            idx = jnp.full((L,), 0, jnp.int32) + i * L + iota
            svb[...] = svb[...] + m
            gtb[...] = gtb[...] + (m > mt).astype(jnp.int32)
            eqb[...] = eqb[...] + ((m == mt) & (idx < tb)).astype(jnp.int32)
            return 0

        lax.fori_loop(0, nz, red, 0)
        total = jnp.sum(svb[...])
        rank = jnp.sum(gtb[...]) + jnp.sum(eqb[...])
        pv = jnp.maximum(jnp.float32(THR), mt / jnp.maximum(jnp.float32(THR), total))
        st_p[...] = pv
        st_r[...] = jnp.full((L,), 0, jnp.int32) + rank
        pltpu.sync_copy(st_p, pt_out.at[pl.ds(b * L, L)])
        pltpu.sync_copy(st_r, rk_out.at[pl.ds(b * L, L)])


def kernel(query, t, s, attention, edge_index, edge_val):
    del s
    qh = query[:, 0].astype(jnp.int32)
    qr = query[:, 1].astype(jnp.int32)
    tt = t.astype(jnp.int32)
    ei = edge_index.astype(jnp.int32).reshape(-1)
    ev = edge_val.astype(jnp.float32).reshape(-1)
    attp = jnp.zeros((B, ATTP), jnp.float32)
    attp = attp.at[:, : BODY_LEN * 17].set(
        attention.astype(jnp.float32).reshape(B, BODY_LEN * 17))
    pk, mv = _mask_pack(ei, ev, qh, qr, tt)
    pt, rk = _propagate(pk, mv, qh, tt, attp.reshape(-1))
    p = pt.reshape(B, L)[:, 0]
    ranks = rk.reshape(B, L)[:, 0]
    batch_loss = jnp.mean(-jnp.log(p))
    return batch_loss, ranks


# speedup vs baseline: 1.0834x; 1.0834x over previous
import functools

import jax
import jax.numpy as jnp
from jax import lax
from jax.experimental import pallas as pl
from jax.experimental.pallas import tpu as pltpu
from jax.experimental.pallas import tpu_sc as plsc

N_ENT = 50000
NUM_POS_REL = 8
BODY_LEN = 3
B = 64
E = 200000
THR = 1e-20

L = 16
NW = 32
C1 = 10000
C2 = 4000
ATTP = 64

_mesh = plsc.VectorSubcoreMesh(core_axis_name="c", subcore_axis_name="s")


def _wid():
    return lax.axis_index("s") * 2 + lax.axis_index("c")


@functools.partial(
    pl.kernel,
    out_type=(
        jax.ShapeDtypeStruct((NUM_POS_REL * E,), jnp.int32),
        jax.ShapeDtypeStruct((NUM_POS_REL * E,), jnp.float32),
    ),
    mesh=_mesh,
    compiler_params=pltpu.CompilerParams(needs_layout_passes=False),
    scratch_types=[
        pltpu.VMEM((B,), jnp.int32),
        pltpu.VMEM((B,), jnp.int32),
        pltpu.VMEM((B,), jnp.int32),
        pltpu.VMEM((B + 2 * L,), jnp.int32),
        pltpu.VMEM((C1,), jnp.int32),
        pltpu.VMEM((C1,), jnp.int32),
        pltpu.VMEM((C1,), jnp.float32),
        pltpu.VMEM((C1,), jnp.int32),
        pltpu.VMEM((C1,), jnp.float32),
        pltpu.VMEM((L,), jnp.int32),
    ],
)
def _mask_pack(ei_hbm, ev_hbm, qh_hbm, qr_hbm, tt_hbm, pk_out, mv_out,
               qh_v, qr_v, tt_v, keyb, srcb, dstb, valb, pkob, mvob, hitb):
    w = _wid()
    rel = w // 4
    quarter = w % 4
    epw = E // 4
    ebase = quarter * epw
    pltpu.sync_copy(qh_hbm, qh_v)
    pltpu.sync_copy(qr_hbm, qr_v)
    pltpu.sync_copy(tt_hbm, tt_v)

    for g in range((B + 2 * L) // L):
        keyb[pl.ds(g * L, L)] = jnp.full((L,), -1, jnp.int32)

    def build(g, cnt):
        qh = qh_v[pl.ds(g * L, L)]
        qr = qr_v[pl.ds(g * L, L)]
        tt = tt_v[pl.ds(g * L, L)]
        valid = qr == rel
        key = jnp.bitwise_or(qh, lax.shift_left(tt, 16))
        vi = valid.astype(jnp.int32)
        pos = cnt + lax.cumsum(vi, axis=0) - 1
        plsc.store_scatter(keyb, [pos], key, mask=valid)
        return cnt + jnp.sum(vi)

    nq = lax.fori_loop(0, B // L, build, jnp.int32(0))

    def chunk(c, _):
        off = ebase + c * C1
        pltpu.sync_copy(ei_hbm.at[pl.ds(rel * 2 * E + off, C1)], srcb)
        pltpu.sync_copy(ei_hbm.at[pl.ds(rel * 2 * E + E + off, C1)], dstb)
        pltpu.sync_copy(ev_hbm.at[pl.ds(rel * E + off, C1)], valb)

        def grp(g, _):
            sl = pl.ds(g * L, L)
            s = srcb[sl]
            d = dstb[sl]
            v = valb[sl]
            pk = jnp.bitwise_or(s, lax.shift_left(d, 16))

            hitb[...] = jnp.zeros((L,), jnp.int32)

            def keyloop(j, _):
                kj = plsc.load_gather(keyb, [jnp.full((L,), j, jnp.int32)])
                hitb[...] = hitb[...] | (pk == kj).astype(jnp.int32)
                return 0

            lax.fori_loop(0, nq, keyloop, 0)
            pkob[sl] = pk
            mvob[sl] = jnp.where(hitb[...] != 0, jnp.float32(0.0), v)
            return 0

        lax.fori_loop(0, C1 // L, grp, 0)
        pltpu.sync_copy(pkob, pk_out.at[pl.ds(rel * E + off, C1)])
        pltpu.sync_copy(mvob, mv_out.at[pl.ds(rel * E + off, C1)])
        return 0

    lax.fori_loop(0, epw // C1, chunk, 0)


@functools.partial(
    pl.kernel,
    out_type=(
        jax.ShapeDtypeStruct((B * L,), jnp.float32),
        jax.ShapeDtypeStruct((B * L,), jnp.int32),
    ),
    mesh=_mesh,
    compiler_params=pltpu.CompilerParams(needs_layout_passes=False),
    scratch_types=[
        pltpu.VMEM((N_ENT,), jnp.float32),
        pltpu.VMEM((N_ENT,), jnp.float32),
        pltpu.VMEM((C2,), jnp.int32),
        pltpu.VMEM((C2,), jnp.float32),
        pltpu.VMEM((C2,), jnp.int32),
        pltpu.VMEM((C2,), jnp.float32),
        pltpu.VMEM((B,), jnp.int32),
        pltpu.VMEM((B,), jnp.int32),
        pltpu.VMEM((ATTP,), jnp.float32),
        pltpu.VMEM((L,), jnp.float32),
        pltpu.VMEM((L,), jnp.int32),
        pltpu.VMEM((L,), jnp.float32),
        pltpu.VMEM((L,), jnp.int32),
        pltpu.VMEM((L,), jnp.int32),
        pltpu.SemaphoreType.DMA,
        pltpu.SemaphoreType.DMA,
    ],
)
def _propagate(pk_hbm, mv_hbm, qh_hbm, tt_hbm, att_hbm, pt_out, rk_out,
               mem_a, mem_b, pkA, vA, pkB, vB, qh_v, tt_v, att_v,
               st_p, st_r, svb, gtb, eqb, semA, semB):
    w = _wid()
    iota = lax.iota(jnp.int32, L)
    lane0 = iota == 0
    nz = N_ENT // L
    nch = E // C2
    ng = C2 // L
    pltpu.sync_copy(qh_hbm, qh_v)
    pltpu.sync_copy(tt_hbm, tt_v)

    def start(r, c, pkb, vb, sem):
        pltpu.async_copy(pk_hbm.at[pl.ds(r * E + c * C2, C2)], pkb, sem)
        pltpu.async_copy(mv_hbm.at[pl.ds(r * E + c * C2, C2)], vb, sem)

    def wait(pkb, vb, sem):
        pltpu.make_async_copy(pk_hbm.at[pl.ds(0, C2)], pkb, sem).wait()
        pltpu.make_async_copy(mv_hbm.at[pl.ds(0, C2)], vb, sem).wait()

    for p in range(2):
        b = w + NW * p
        pltpu.sync_copy(att_hbm.at[pl.ds(b * ATTP, ATTP)], att_v)
        bvec = jnp.full((L,), 0, jnp.int32) + b
        hb = plsc.load_gather(qh_v, [bvec])
        tb = plsc.load_gather(tt_v, [bvec])

        @plsc.parallel_loop(0, nz, unroll=8)
        def zero_a(i):
            mem_a[pl.ds(i * L, L)] = jnp.zeros((L,), jnp.float32)
        plsc.store_scatter(mem_a, [hb], jnp.ones((L,), jnp.float32), mask=lane0)

        bufs = (mem_a, mem_b)
        for step in range(BODY_LEN):
            src = bufs[step % 2]
            dst = bufs[1 - step % 2]

            @plsc.parallel_loop(0, nz, unroll=8)
            def zero_d(i):
                dst[pl.ds(i * L, L)] = jnp.zeros((L,), jnp.float32)

            def startg(g, pkb, vb, sem):
                pltpu.async_copy(pk_hbm.at[pl.ds(g * C2, C2)], pkb, sem)
                pltpu.async_copy(mv_hbm.at[pl.ds(g * C2, C2)], vb, sem)

            def process(g, pkb, vb):
                r = g // nch
                attF = plsc.load_gather(att_v, [jnp.full((L,), step * 17, jnp.int32) + r])
                attB = plsc.load_gather(att_v, [jnp.full((L,), step * 17 + 8, jnp.int32) + r])

                @plsc.parallel_loop(0, ng, unroll=8)
                def grp(q):
                    sl = pl.ds(q * L, L)
                    wv = pkb[sl]
                    vv = vb[sl]
                    sidx = jnp.bitwise_and(wv, jnp.int32(0xFFFF))
                    didx = lax.shift_right_logical(wv, 16)
                    xf = plsc.load_gather(src, [sidx])
                    plsc.addupdate_scatter(dst, [didx], xf * vv * attF)
                    xb = plsc.load_gather(src, [didx])
                    plsc.addupdate_scatter(dst, [sidx], xb * vv * attB)

            ncha = NUM_POS_REL * nch
            startg(0, pkA, vA, semA)

            def pair(i, _):
                startg(2 * i + 1, pkB, vB, semB)
                wait(pkA, vA, semA)
                process(2 * i, pkA, vA)

                @pl.when(i < ncha // 2 - 1)
                def _():
                    startg(2 * i + 2, pkA, vA, semA)

                wait(pkB, vB, semB)
                process(2 * i + 1, pkB, vB)
                return 0

            lax.fori_loop(0, ncha // 2, pair, 0)

            attS = plsc.load_gather(att_v, [jnp.full((L,), step * 17 + 16, jnp.int32)])

            @plsc.parallel_loop(0, nz, unroll=8)
            def selfadd(i):
                sl = pl.ds(i * L, L)
                dst[sl] = dst[sl] + src[sl] * attS

        fin = bufs[BODY_LEN % 2]
        mt = plsc.load_gather(fin, [tb])

        svb[...] = jnp.zeros((L,), jnp.float32)
        gtb[...] = jnp.zeros((L,), jnp.int32)
        eqb[...] = jnp.zeros((L,), jnp.int32)

        def red(i, _):
            m = fin[pl.ds(i * L, L)]
            idx = jnp.full((L,), 0, jnp.int32) + i * L + iota
            svb[...] = svb[...] + m
            gtb[...] = gtb[...] + (m > mt).astype(jnp.int32)
            eqb[...] = eqb[...] + ((m == mt) & (idx < tb)).astype(jnp.int32)
            return 0

        lax.fori_loop(0, nz, red, 0)
        total = jnp.sum(svb[...])
        rank = jnp.sum(gtb[...]) + jnp.sum(eqb[...])
        pv = jnp.maximum(jnp.float32(THR), mt / jnp.maximum(jnp.float32(THR), total))
        st_p[...] = pv
        st_r[...] = jnp.full((L,), 0, jnp.int32) + rank
        pltpu.sync_copy(st_p, pt_out.at[pl.ds(b * L, L)])
        pltpu.sync_copy(st_r, rk_out.at[pl.ds(b * L, L)])


def kernel(query, t, s, attention, edge_index, edge_val):
    del s
    qh = query[:, 0].astype(jnp.int32)
    qr = query[:, 1].astype(jnp.int32)
    tt = t.astype(jnp.int32)
    ei = edge_index.astype(jnp.int32).reshape(-1)
    ev = edge_val.astype(jnp.float32).reshape(-1)
    attp = jnp.zeros((B, ATTP), jnp.float32)
    attp = attp.at[:, : BODY_LEN * 17].set(
        attention.astype(jnp.float32).reshape(B, BODY_LEN * 17))
    pk, mv = _mask_pack(ei, ev, qh, qr, tt)
    pt, rk = _propagate(pk, mv, qh, tt, attp.reshape(-1))
    p = pt.reshape(B, L)[:, 0]
    ranks = rk.reshape(B, L)[:, 0]
    batch_loss = jnp.mean(-jnp.log(p))
    return batch_loss, ranks
